# baseline (speedup 1.0000x reference)
import jax
import jax.numpy as jnp
from jax.experimental import pallas as pl
from jax.experimental.pallas import tpu as pltpu

_N = 54
_E = 2862
_EP = 2944
_D_IN, _D_HID, _D_OUT, _D_GLOB = 512, 512, 256, 64
_EMB = _N * _D_OUT
_WO1R = _EMB + _D_GLOB

_OB1, _OB2, _OB3 = 0, 512, 1024
_OBG1, _OBG2, _OBG3 = 1280, 1408, 1536
_OBO1, _OBO2, _OGLB, _OWO2 = 1664, 1792, 1920, 2048
_PACK = 2176


def _net_kernel(ei_ref, pack_ref, wg_ref, feat_ref,
                W1_hbm, W2_hbm, W3_hbm, Wo1_hbm, out_ref,
                w1_s, w2_s, w3_s, wo1_s, s1, s2, s3, s4):
    f32 = jnp.float32
    cp1 = pltpu.make_async_copy(W1_hbm, w1_s, s1)
    cp1.start()
    cp2 = pltpu.make_async_copy(W2_hbm, w2_s, s2)
    cp2.start()
    cp3 = pltpu.make_async_copy(W3_hbm, w3_s, s3)
    cp3.start()
    cp4 = pltpu.make_async_copy(Wo1_hbm, wo1_s, s4)
    cp4.start()

    src = ei_ref[0:1, :]
    dst = ei_ref[1:2, :]
    node_iota = jax.lax.broadcasted_iota(jnp.int32, (_N, _EP), 0)
    oh_src = (src == node_iota).astype(f32)
    oh_dst = (dst == node_iota).astype(f32)
    A = jax.lax.dot_general(oh_dst, oh_src, (((1,), (1,)), ((), ())),
                            preferred_element_type=f32)
    deg_out = jnp.sum(A, axis=0, keepdims=True)
    deg_in = jnp.sum(A, axis=1, keepdims=True)
    n_out = jax.lax.rsqrt(jnp.maximum(deg_out, 1.0))
    n_in = jax.lax.rsqrt(jnp.maximum(deg_in, 1.0))
    Ahat = A * n_in * n_out

    pack = pack_ref[...]
    g = pack[:, _OGLB:_OGLB + _D_GLOB]
    g = jnp.maximum(jnp.dot(g, wg_ref[:, 0:16]) + pack[:, _OBG1:_OBG1 + 16],
                    0.0)
    g = jnp.maximum(jnp.dot(g, wg_ref[0:16, 128:144])
                    + pack[:, _OBG2:_OBG2 + 16], 0.0)
    g = jnp.maximum(jnp.dot(g, wg_ref[0:16, 256:320])
                    + pack[:, _OBG3:_OBG3 + _D_GLOB], 0.0)

    ax = jnp.dot(Ahat, feat_ref[...], preferred_element_type=f32)
    cp1.wait()
    h = jnp.maximum(jnp.dot(ax, w1_s[...], preferred_element_type=f32)
                    + pack[:, _OB1:_OB1 + _D_HID], 0.0)
    ah = jnp.dot(Ahat, h, preferred_element_type=f32)
    cp2.wait()
    h = jnp.maximum(jnp.dot(ah, w2_s[...], preferred_element_type=f32)
                    + pack[:, _OB2:_OB2 + _D_HID], 0.0)
    ah = jnp.dot(Ahat, h, preferred_element_type=f32)
    cp3.wait()
    emb = jnp.maximum(jnp.dot(ah, w3_s[...], preferred_element_type=f32)
                      + pack[:, _OB3:_OB3 + _D_OUT], 0.0)

    emb_flat = emb.reshape(1, _EMB)
    cp4.wait()
    out1 = (jnp.dot(emb_flat, wo1_s[0:_EMB, :], preferred_element_type=f32)
            + jnp.dot(g, wo1_s[_EMB:_WO1R, :], preferred_element_type=f32)
            + pack[:, _OBO1:_OBO1 + 85])
    out1 = jnp.maximum(out1, 0.0)
    out2 = (jnp.sum(out1 * pack[:, _OWO2:_OWO2 + 85], axis=1, keepdims=True)
            + pack[:, _OBO2:_OBO2 + 1])
    out_ref[...] = jax.nn.sigmoid(out2)


def kernel(feat, edge_index, globalFeats, isTrain,
           W1, b1, W2, b2, W3, b3,
           Wg1, bg1, Wg2, bg2, Wg3, bg3,
           Wo1, bo1, Wo2, bo2):
    f32 = jnp.float32
    ei = jnp.concatenate(
        [edge_index.astype(jnp.int32),
         jnp.full((2, _EP - _E), jnp.int32(1 << 20), dtype=jnp.int32)], axis=1)

    def pad_to(v, n):
        return jnp.pad(v.reshape(-1), (0, n - v.size))

    pack = jnp.concatenate([
        b1, b2, pad_to(b3, 256),
        pad_to(bg1, 128), pad_to(bg2, 128), pad_to(bg3, 128),
        pad_to(bo1, 128), pad_to(bo2, 128),
        pad_to(globalFeats, 128), pad_to(Wo2, 128),
    ]).reshape(1, _PACK)
    wg = jnp.zeros((64, 384), dtype=f32)
    wg = jax.lax.dynamic_update_slice(wg, Wg1, (0, 0))
    wg = jax.lax.dynamic_update_slice(wg, Wg2, (0, 128))
    wg = jax.lax.dynamic_update_slice(wg, Wg3, (0, 256))

    vmem = pl.BlockSpec(memory_space=pltpu.MemorySpace.VMEM)
    hbm = pl.BlockSpec(memory_space=pltpu.MemorySpace.HBM)
    out = pl.pallas_call(
        _net_kernel,
        out_shape=jax.ShapeDtypeStruct((1, 1), f32),
        in_specs=[vmem, vmem, vmem, vmem, hbm, hbm, hbm, hbm],
        out_specs=vmem,
        scratch_shapes=[
            pltpu.VMEM((_D_IN, _D_HID), f32),
            pltpu.VMEM((_D_HID, _D_HID), f32),
            pltpu.VMEM((_D_HID, _D_OUT), f32),
            pltpu.VMEM((_WO1R, 85), f32),
            pltpu.SemaphoreType.DMA,
            pltpu.SemaphoreType.DMA,
            pltpu.SemaphoreType.DMA,
            pltpu.SemaphoreType.DMA,
        ],
    )(ei, pack, wg, feat, W1, W2, W3, Wo1)
    return out.reshape(1)


# speedup vs baseline: 1.4365x; 1.4365x over previous
import jax
import jax.numpy as jnp
from jax.experimental import pallas as pl
from jax.experimental.pallas import tpu as pltpu

_N = 54
_E = 2862
_EP = 2944
_D_IN, _D_HID, _D_OUT, _D_GLOB = 512, 512, 256, 64
_EMB = _N * _D_OUT
_CH = 4608


def _net_kernel(src_ref, dst_ref, feat_ref, glob_ref,
                W1_hbm, b1_ref, W2_hbm, b2_ref, W3_hbm, b3_ref,
                Wg1_ref, bg1_ref, Wg2_ref, bg2_ref, Wg3_ref, bg3_ref,
                Wo1_hbm, bo1_ref, Wo2_ref, bo2_ref, out_ref,
                w1_s, w2_s, w3_s, c0_s, c1_s, c2_s, cg_s,
                s1, s2, s3, sc0, sc1, sc2, scg):
    f32 = jnp.float32
    cp1 = pltpu.make_async_copy(W1_hbm, w1_s, s1)
    cp1.start()
    cp2 = pltpu.make_async_copy(W2_hbm, w2_s, s2)
    cp2.start()
    cp3 = pltpu.make_async_copy(W3_hbm, w3_s, s3)
    cp3.start()
    cc0 = pltpu.make_async_copy(Wo1_hbm.at[pl.ds(0, _CH), :], c0_s, sc0)
    cc0.start()
    cc1 = pltpu.make_async_copy(Wo1_hbm.at[pl.ds(_CH, _CH), :], c1_s, sc1)
    cc1.start()
    cc2 = pltpu.make_async_copy(Wo1_hbm.at[pl.ds(2 * _CH, _CH), :], c2_s, sc2)
    cc2.start()
    ccg = pltpu.make_async_copy(Wo1_hbm.at[pl.ds(_EMB, _D_GLOB), :], cg_s, scg)
    ccg.start()

    scope = jax.named_scope
    src = src_ref[...]
    dst = dst_ref[...]
    with scope("adjbuild"):
        node_iota = jax.lax.broadcasted_iota(jnp.int32, (_N, _EP), 0)
        oh_src = (src == node_iota).astype(f32)
        oh_dst = (dst == node_iota).astype(f32)
        A = jax.lax.dot_general(oh_dst, oh_src, (((1,), (1,)), ((), ())),
                                preferred_element_type=f32)
        deg_out = jnp.sum(A, axis=0, keepdims=True)
        deg_in = jnp.sum(A, axis=1, keepdims=True)
        n_out = jax.lax.rsqrt(jnp.maximum(deg_out, 1.0))
        n_in = jax.lax.rsqrt(jnp.maximum(deg_in, 1.0))
        Ahat = A * n_in * n_out

    with scope("globmlp"):
        g = glob_ref[...]
        g = jnp.maximum(jnp.dot(g, Wg1_ref[...]) + bg1_ref[...], 0.0)
        g = jnp.maximum(jnp.dot(g, Wg2_ref[...]) + bg2_ref[...], 0.0)
        g = jnp.maximum(jnp.dot(g, Wg3_ref[...]) + bg3_ref[...], 0.0)

    with scope("conv1"):
        ax = jnp.dot(Ahat, feat_ref[...], preferred_element_type=f32)
        cp1.wait()
        h = jnp.maximum(jnp.dot(ax, w1_s[...], preferred_element_type=f32)
                        + b1_ref[...], 0.0)
    with scope("conv2"):
        ah = jnp.dot(Ahat, h, preferred_element_type=f32)
        cp2.wait()
        h = jnp.maximum(jnp.dot(ah, w2_s[...], preferred_element_type=f32)
                        + b2_ref[...], 0.0)
    with scope("conv3"):
        ah = jnp.dot(Ahat, h, preferred_element_type=f32)
        cp3.wait()
        emb = jnp.maximum(jnp.dot(ah, w3_s[...], preferred_element_type=f32)
                          + b3_ref[...], 0.0)

    with scope("reshape"):
        emb_flat = emb.reshape(1, _EMB)
    cc0.wait()
    cc1.wait()
    cc2.wait()
    ccg.wait()
    out1 = (jnp.dot(emb_flat[:, :_CH], c0_s[...], preferred_element_type=f32)
            + jnp.dot(emb_flat[:, _CH:2 * _CH], c1_s[...],
                      preferred_element_type=f32)
            + jnp.dot(emb_flat[:, 2 * _CH:], c2_s[...],
                      preferred_element_type=f32)
            + jnp.dot(g, cg_s[...], preferred_element_type=f32)
            + bo1_ref[...])
    out1 = jnp.maximum(out1, 0.0)
    out2 = (jnp.dot(out1, Wo2_ref[...], preferred_element_type=f32)
            + bo2_ref[...])
    out_ref[...] = jax.nn.sigmoid(out2)


def kernel(feat, edge_index, globalFeats, isTrain,
           W1, b1, W2, b2, W3, b3,
           Wg1, bg1, Wg2, bg2, Wg3, bg3,
           Wo1, bo1, Wo2, bo2):
    ei = jnp.concatenate(
        [edge_index.astype(jnp.int32),
         jnp.full((2, _EP - _E), jnp.int32(1 << 20), dtype=jnp.int32)], axis=1)
    src = ei[0].reshape(1, _EP)
    dst = ei[1].reshape(1, _EP)
    glob = globalFeats.reshape(1, _D_GLOB)
    f32 = jnp.float32
    vmem = pl.BlockSpec(memory_space=pltpu.MemorySpace.VMEM)
    hbm = pl.BlockSpec(memory_space=pltpu.MemorySpace.HBM)
    out = pl.pallas_call(
        _net_kernel,
        out_shape=jax.ShapeDtypeStruct((1, 1), f32),
        in_specs=[vmem, vmem, vmem, vmem,
                  hbm, vmem, hbm, vmem, hbm, vmem,
                  vmem, vmem, vmem, vmem, vmem, vmem,
                  hbm, vmem, vmem, vmem],
        out_specs=vmem,
        scratch_shapes=[
            pltpu.VMEM((_D_IN, _D_HID), f32),
            pltpu.VMEM((_D_HID, _D_HID), f32),
            pltpu.VMEM((_D_HID, _D_OUT), f32),
            pltpu.VMEM((_CH, 85), f32),
            pltpu.VMEM((_CH, 85), f32),
            pltpu.VMEM((_CH, 85), f32),
            pltpu.VMEM((_D_GLOB, 85), f32),
            pltpu.SemaphoreType.DMA,
            pltpu.SemaphoreType.DMA,
            pltpu.SemaphoreType.DMA,
            pltpu.SemaphoreType.DMA,
            pltpu.SemaphoreType.DMA,
            pltpu.SemaphoreType.DMA,
            pltpu.SemaphoreType.DMA,
        ],
    )(src, dst, feat, glob,
      W1, b1.reshape(1, -1), W2, b2.reshape(1, -1), W3, b3.reshape(1, -1),
      Wg1, bg1.reshape(1, -1), Wg2, bg2.reshape(1, -1), Wg3, bg3.reshape(1, -1),
      Wo1, bo1.reshape(1, -1), Wo2, bo2.reshape(1, -1))
    return out.reshape(1)


# speedup vs baseline: 1.5706x; 1.0934x over previous
import jax
import jax.numpy as jnp
from jax.experimental import pallas as pl
from jax.experimental.pallas import tpu as pltpu

_N = 54
_E = 2862
_D_IN, _D_HID, _D_OUT, _D_GLOB = 512, 512, 256, 64
_EMB = _N * _D_OUT
_WO1R = _EMB + _D_GLOB


def _net_kernel(ei_ref, feat_ref, glob_ref,
                W1_hbm, b1_ref, W2_hbm, b2_ref, W3_hbm, b3_ref,
                Wg1_ref, bg1_ref, Wg2_ref, bg2_ref, Wg3_ref, bg3_ref,
                Wo1_hbm, bo1_ref, Wo2_ref, bo2_ref, out_ref,
                w1_s, w2_s, w3_s, wo1_s, s1, s2, s3, s4):
    f32 = jnp.float32
    cp1 = pltpu.make_async_copy(W1_hbm, w1_s, s1)
    cp1.start()
    cp2 = pltpu.make_async_copy(W2_hbm, w2_s, s2)
    cp2.start()
    cp3 = pltpu.make_async_copy(W3_hbm, w3_s, s3)
    cp3.start()
    cp4 = pltpu.make_async_copy(Wo1_hbm, wo1_s, s4)
    cp4.start()

    src = ei_ref[0:1, :]
    dst = ei_ref[1:2, :]
    node_iota = jax.lax.broadcasted_iota(jnp.int32, (_N, _E), 0)
    oh_src = (src == node_iota).astype(f32)
    oh_dst = (dst == node_iota).astype(f32)
    A = jax.lax.dot_general(oh_dst, oh_src, (((1,), (1,)), ((), ())),
                            preferred_element_type=f32)
    deg_out = jnp.sum(A, axis=0, keepdims=True)
    deg_in = jnp.sum(A, axis=1, keepdims=True)
    n_out = jax.lax.rsqrt(jnp.maximum(deg_out, 1.0))
    n_in = jax.lax.rsqrt(jnp.maximum(deg_in, 1.0))
    Ahat = A * n_in * n_out

    g = glob_ref[...].reshape(1, _D_GLOB)
    g = jnp.maximum(jnp.dot(g, Wg1_ref[...]) + bg1_ref[...], 0.0)
    g = jnp.maximum(jnp.dot(g, Wg2_ref[...]) + bg2_ref[...], 0.0)
    g = jnp.maximum(jnp.dot(g, Wg3_ref[...]) + bg3_ref[...], 0.0)

    ax = jnp.dot(Ahat, feat_ref[...], preferred_element_type=f32)
    cp1.wait()
    h = jnp.maximum(jnp.dot(ax, w1_s[...], preferred_element_type=f32)
                    + b1_ref[...], 0.0)
    ah = jnp.dot(Ahat, h, preferred_element_type=f32)
    cp2.wait()
    h = jnp.maximum(jnp.dot(ah, w2_s[...], preferred_element_type=f32)
                    + b2_ref[...], 0.0)
    ah = jnp.dot(Ahat, h, preferred_element_type=f32)
    cp3.wait()
    emb = jnp.maximum(jnp.dot(ah, w3_s[...], preferred_element_type=f32)
                      + b3_ref[...], 0.0)

    cat = jnp.concatenate([emb.reshape(1, _EMB), g], axis=1)
    cp4.wait()
    out1 = (jnp.dot(cat, wo1_s[...], preferred_element_type=f32)
            + bo1_ref[...])
    out1 = jnp.maximum(out1, 0.0)
    out2 = (jnp.dot(out1, Wo2_ref[...], preferred_element_type=f32)
            + bo2_ref[...])
    out_ref[...] = jax.nn.sigmoid(out2)


def kernel(feat, edge_index, globalFeats, isTrain,
           W1, b1, W2, b2, W3, b3,
           Wg1, bg1, Wg2, bg2, Wg3, bg3,
           Wo1, bo1, Wo2, bo2):
    f32 = jnp.float32
    vmem = pl.BlockSpec(memory_space=pltpu.MemorySpace.VMEM)
    hbm = pl.BlockSpec(memory_space=pltpu.MemorySpace.HBM)
    out = pl.pallas_call(
        _net_kernel,
        out_shape=jax.ShapeDtypeStruct((1, 1), f32),
        in_specs=[vmem, vmem, vmem,
                  hbm, vmem, hbm, vmem, hbm, vmem,
                  vmem, vmem, vmem, vmem, vmem, vmem,
                  hbm, vmem, vmem, vmem],
        out_specs=vmem,
        scratch_shapes=[
            pltpu.VMEM((_D_IN, _D_HID), f32),
            pltpu.VMEM((_D_HID, _D_HID), f32),
            pltpu.VMEM((_D_HID, _D_OUT), f32),
            pltpu.VMEM((_WO1R, 85), f32),
            pltpu.SemaphoreType.DMA,
            pltpu.SemaphoreType.DMA,
            pltpu.SemaphoreType.DMA,
            pltpu.SemaphoreType.DMA,
        ],
    )(edge_index.astype(jnp.int32), feat, globalFeats,
      W1, b1, W2, b2, W3, b3,
      Wg1, bg1, Wg2, bg2, Wg3, bg3,
      Wo1, bo1, Wo2, bo2)
    return out.reshape(1)


# speedup vs baseline: 4.1022x; 2.6119x over previous
import jax
import jax.numpy as jnp
from jax.experimental import pallas as pl
from jax.experimental.pallas import tpu as pltpu

_N = 54
_E = 2862
_D_IN, _D_HID, _D_OUT, _D_GLOB = 512, 512, 256, 64
_EMB = _N * _D_OUT
_WO1R = _EMB + _D_GLOB


def _net_kernel(ei_ref, feat_ref, glob_ref,
                W1_hbm, b1_ref, W2_hbm, b2_ref, W3_hbm, b3_ref,
                Wg1T_ref, bg1_ref, Wg2_ref, bg2_ref, Wg3_ref, bg3_ref,
                Wo1T_hbm, bo1_ref, Wo2T_ref, bo2_ref, out_ref,
                w1_s, w2_s, w3_s, wo1t_s, s1, s2, s3, s4):
    f32 = jnp.float32
    cp1 = pltpu.make_async_copy(W1_hbm, w1_s, s1)
    cp1.start()
    cp2 = pltpu.make_async_copy(W2_hbm, w2_s, s2)
    cp2.start()
    cp3 = pltpu.make_async_copy(W3_hbm, w3_s, s3)
    cp3.start()
    cp4 = pltpu.make_async_copy(Wo1T_hbm, wo1t_s, s4)
    cp4.start()

    src = ei_ref[0:1, :]
    dst = ei_ref[1:2, :]
    node_iota = jax.lax.broadcasted_iota(jnp.int32, (_N, _E), 0)
    oh_src = (src == node_iota).astype(f32)
    oh_dst = (dst == node_iota).astype(f32)
    A = jax.lax.dot_general(oh_dst, oh_src, (((1,), (1,)), ((), ())),
                            preferred_element_type=f32)
    deg_out = jnp.sum(A, axis=0, keepdims=True)
    deg_in = jnp.sum(A, axis=1, keepdims=True)
    n_out = jax.lax.rsqrt(jnp.maximum(deg_out, 1.0))
    n_in = jax.lax.rsqrt(jnp.maximum(deg_in, 1.0))
    Ahat = A * n_in * n_out

    g = glob_ref[...].reshape(1, _D_GLOB)
    g = jnp.maximum(
        jax.lax.dot_general(g, Wg1T_ref[...], (((1,), (1,)), ((), ())),
                            preferred_element_type=f32) + bg1_ref[...], 0.0)
    g = jnp.maximum(jnp.dot(g, Wg2_ref[...]) + bg2_ref[...], 0.0)
    g = jnp.maximum(jnp.dot(g, Wg3_ref[...]) + bg3_ref[...], 0.0)

    ax = jnp.dot(Ahat, feat_ref[...], preferred_element_type=f32)
    cp1.wait()
    h = jnp.maximum(jnp.dot(ax, w1_s[...], preferred_element_type=f32)
                    + b1_ref[...], 0.0)
    ah = jnp.dot(Ahat, h, preferred_element_type=f32)
    cp2.wait()
    h = jnp.maximum(jnp.dot(ah, w2_s[...], preferred_element_type=f32)
                    + b2_ref[...], 0.0)
    ah = jnp.dot(Ahat, h, preferred_element_type=f32)
    cp3.wait()
    emb = jnp.maximum(jnp.dot(ah, w3_s[...], preferred_element_type=f32)
                      + b3_ref[...], 0.0)

    cat = jnp.concatenate([emb.reshape(1, _EMB), g], axis=1)
    cp4.wait()
    out1 = (jax.lax.dot_general(cat, wo1t_s[...], (((1,), (1,)), ((), ())),
                                preferred_element_type=f32)
            + bo1_ref[...])
    out1 = jnp.maximum(out1, 0.0)
    out2 = (jnp.sum(out1 * Wo2T_ref[...], axis=1, keepdims=True)
            + bo2_ref[...])
    out_ref[...] = jax.nn.sigmoid(out2)


def kernel(feat, edge_index, globalFeats, isTrain,
           W1, b1, W2, b2, W3, b3,
           Wg1, bg1, Wg2, bg2, Wg3, bg3,
           Wo1, bo1, Wo2, bo2):
    f32 = jnp.float32
    vmem = pl.BlockSpec(memory_space=pltpu.MemorySpace.VMEM)
    hbm = pl.BlockSpec(memory_space=pltpu.MemorySpace.HBM)
    out = pl.pallas_call(
        _net_kernel,
        out_shape=jax.ShapeDtypeStruct((1, 1), f32),
        in_specs=[vmem, vmem, vmem,
                  hbm, vmem, hbm, vmem, hbm, vmem,
                  vmem, vmem, vmem, vmem, vmem, vmem,
                  hbm, vmem, vmem, vmem],
        out_specs=vmem,
        scratch_shapes=[
            pltpu.VMEM((_D_IN, _D_HID), f32),
            pltpu.VMEM((_D_HID, _D_HID), f32),
            pltpu.VMEM((_D_HID, _D_OUT), f32),
            pltpu.VMEM((85, _WO1R), f32),
            pltpu.SemaphoreType.DMA,
            pltpu.SemaphoreType.DMA,
            pltpu.SemaphoreType.DMA,
            pltpu.SemaphoreType.DMA,
        ],
    )(edge_index.astype(jnp.int32), feat, globalFeats,
      W1, b1, W2, b2, W3, b3,
      Wg1.T, bg1, Wg2, bg2, Wg3, bg3,
      Wo1.T, bo1, Wo2.T, bo2)
    return out.reshape(1)
